# Initial kernel scaffold; baseline (speedup 1.0000x reference)
import jax
import jax.numpy as jnp
from jax.experimental import pallas as pl


def kernel(inputs):
    raise NotImplementedError("write your pallas kernel here")


# speedup vs baseline: 1.1997x; 1.1997x over previous
import jax
import jax.numpy as jnp
from jax.experimental import pallas as pl
from jax.experimental.pallas import tpu as pltpu

_B, _S, _F = 8, 1024, 256


def _body(in_ref, out_ref, minval):
    i = pl.program_id(0)

    @pl.when(i < _B)
    def _copy_and_reduce():
        x = in_ref[...]
        bmin = jnp.min(x)
        prev = jnp.where(i == 0, bmin, minval[0])
        minval[0] = jnp.minimum(prev, bmin)
        out_ref[...] = x

    @pl.when(i >= _B)
    def _fill():
        out_ref[...] = jnp.full(out_ref.shape, minval[0] - 1.0, out_ref.dtype)


def kernel(inputs):
    grid = (2 * _B,)
    out = pl.pallas_call(
        _body,
        grid=grid,
        in_specs=[
            pl.BlockSpec((1, _S, _F), lambda i: (jnp.minimum(i, _B - 1), 0, 0)),
        ],
        out_specs=pl.BlockSpec(
            (1, _S, _F), lambda i: (jnp.mod(i, _B), i // _B, 0)
        ),
        out_shape=jax.ShapeDtypeStruct((_B, 2 * _S, _F), inputs.dtype),
        scratch_shapes=[pltpu.SMEM((1,), jnp.float32)],
    )(inputs)
    return out


# speedup vs baseline: 2.0454x; 1.7049x over previous
import jax
import jax.numpy as jnp
from jax.experimental import pallas as pl
from jax.experimental.pallas import tpu as pltpu

_B, _S, _F = 8, 1024, 256


def _body(in_hbm, out_hbm, stage, fillbuf, in_sems, out_sems, fill_sems):
    for b in range(_B):
        pltpu.make_async_copy(in_hbm.at[b], stage.at[b], in_sems.at[b]).start()

    minval = None
    for b in range(_B):
        pltpu.make_async_copy(in_hbm.at[b], stage.at[b], in_sems.at[b]).wait()
        pltpu.make_async_copy(
            stage.at[b], out_hbm.at[b, 0:_S], out_sems.at[b]
        ).start()
        bmin = jnp.min(stage[b])
        minval = bmin if minval is None else jnp.minimum(minval, bmin)

    fillbuf[...] = jnp.full(fillbuf.shape, minval - 1.0, fillbuf.dtype)
    for b in range(_B):
        pltpu.make_async_copy(
            fillbuf, out_hbm.at[b, _S : 2 * _S], fill_sems.at[b]
        ).start()

    for b in range(_B):
        pltpu.make_async_copy(
            stage.at[b], out_hbm.at[b, 0:_S], out_sems.at[b]
        ).wait()
        pltpu.make_async_copy(
            fillbuf, out_hbm.at[b, _S : 2 * _S], fill_sems.at[b]
        ).wait()


def kernel(inputs):
    return pl.pallas_call(
        _body,
        in_specs=[pl.BlockSpec(memory_space=pltpu.MemorySpace.HBM)],
        out_specs=pl.BlockSpec(memory_space=pltpu.MemorySpace.HBM),
        out_shape=jax.ShapeDtypeStruct((_B, 2 * _S, _F), inputs.dtype),
        scratch_shapes=[
            pltpu.VMEM((_B, _S, _F), jnp.float32),
            pltpu.VMEM((_S, _F), jnp.float32),
            pltpu.SemaphoreType.DMA((_B,)),
            pltpu.SemaphoreType.DMA((_B,)),
            pltpu.SemaphoreType.DMA((_B,)),
        ],
    )(inputs)
